# Initial kernel scaffold; baseline (speedup 1.0000x reference)
import jax
import jax.numpy as jnp
from jax.experimental import pallas as pl


def kernel(categorical, continuous, tables, W1, b1, W2, b2, W3, b3):
    raise NotImplementedError("write your pallas kernel here")


# speedup vs baseline: 2.0099x; 2.0099x over previous
import functools

import jax
import jax.numpy as jnp
from jax import lax
from jax.experimental import pallas as pl
from jax.experimental.pallas import tpu as pltpu
from jax.experimental.pallas import tpu_sc as plsc

N_FIELDS = 26
VOCAB = 100000
EMB_DIM = 16
N_CONT = 13
BATCH = 16384
H1, H2 = 64, 32

NC = 2
NS = 16
NW = NC * NS
BPW = BATCH // NW

IN_PAD = 432


def _sc_gather(cat_t, tables):
    mesh = plsc.VectorSubcoreMesh(core_axis_name="c", subcore_axis_name="s")

    @functools.partial(
        pl.kernel,
        out_type=jax.ShapeDtypeStruct((N_FIELDS, BATCH, EMB_DIM), jnp.float32),
        mesh=mesh,
        scratch_types=[
            pltpu.VMEM((N_FIELDS, BPW), jnp.int32),
            pltpu.VMEM((2, BPW, EMB_DIM), jnp.float32),
            pltpu.SemaphoreType.DMA,
            pltpu.SemaphoreType.DMA,
        ],
        compiler_params=pltpu.CompilerParams(use_tc_tiling_on_sc=False),
    )
    def k(cat_hbm, *rest):
        tables_hbm = rest[:N_FIELDS]
        out_hbm, idx_v, rows_v, gsem, wsem = rest[N_FIELDS:]
        wid = lax.axis_index("s") * NC + lax.axis_index("c")
        base = wid * BPW
        pltpu.sync_copy(cat_hbm.at[:, pl.ds(base, BPW)], idx_v)
        g = pltpu.async_copy(tables_hbm[0].at[idx_v.at[0]], rows_v.at[0], gsem)
        w = None
        for f in range(N_FIELDS):
            g.wait()
            if w is not None:
                w.wait()
            w = pltpu.async_copy(
                rows_v.at[f % 2], out_hbm.at[f, pl.ds(base, BPW)], wsem
            )
            if f + 1 < N_FIELDS:
                g = pltpu.async_copy(
                    tables_hbm[f + 1].at[idx_v.at[f + 1]],
                    rows_v.at[(f + 1) % 2], gsem,
                )
        w.wait()

    return k(cat_t, *tables)


def _mlp(emb, cont, w1p, b1, w2, b2, w3, b3):
    bb = 1024
    grid = (BATCH // bb,)

    def body(emb_ref, cont_ref, w1_ref, b1_ref, w2_ref, b2_ref, w3_ref,
             b3_ref, out_ref):
        pieces = [emb_ref[f] for f in range(N_FIELDS)]
        pieces.append(cont_ref[:])
        pieces.append(jnp.zeros((bb, IN_PAD - N_FIELDS * EMB_DIM - N_CONT),
                                jnp.float32))
        x = jnp.concatenate(pieces, axis=1)
        h = jnp.dot(x, w1_ref[:], preferred_element_type=jnp.float32)
        h = jnp.maximum(h + b1_ref[:], 0.0)
        h = jnp.dot(h, w2_ref[:], preferred_element_type=jnp.float32)
        h = jnp.maximum(h + b2_ref[:], 0.0)
        out = jnp.dot(h, w3_ref[:], preferred_element_type=jnp.float32)
        out_ref[:] = out[:, 0] + b3_ref[0]

    return pl.pallas_call(
        body,
        grid=grid,
        in_specs=[
            pl.BlockSpec((N_FIELDS, bb, EMB_DIM), lambda i: (0, i, 0)),
            pl.BlockSpec((bb, N_CONT), lambda i: (i, 0)),
            pl.BlockSpec((IN_PAD, H1), lambda i: (0, 0)),
            pl.BlockSpec((H1,), lambda i: (0,)),
            pl.BlockSpec((H1, H2), lambda i: (0, 0)),
            pl.BlockSpec((H2,), lambda i: (0,)),
            pl.BlockSpec((H2, 1), lambda i: (0, 0)),
            pl.BlockSpec((1,), lambda i: (0,)),
        ],
        out_specs=pl.BlockSpec((bb,), lambda i: (i,)),
        out_shape=jax.ShapeDtypeStruct((BATCH,), jnp.float32),
        compiler_params=pltpu.CompilerParams(vmem_limit_bytes=100 * 1024 * 1024),
    )(emb, cont, w1p, b1, w2, b2, w3, b3)


def kernel(categorical, continuous, tables, W1, b1, W2, b2, W3, b3):
    cat_t = categorical.T
    emb = _sc_gather(cat_t, tables)
    w1p = jnp.zeros((IN_PAD, H1), jnp.float32).at[: W1.shape[0]].set(W1)
    return _mlp(emb, continuous, w1p, b1, W2, b2, W3, b3)


# speedup vs baseline: 3.2916x; 1.6377x over previous
import functools

import jax
import jax.numpy as jnp
from jax import lax
from jax.experimental import pallas as pl
from jax.experimental.pallas import tpu as pltpu
from jax.experimental.pallas import tpu_sc as plsc

N_FIELDS = 26
VOCAB = 100000
EMB_DIM = 16
N_CONT = 13
BATCH = 16384
H1, H2 = 64, 32

NC = 2
NS = 16
NW = NC * NS
BPW = BATCH // NW

IN_PAD = 432
GLEN = BPW * EMB_DIM


def _sc_gather(catf, flats):
    mesh = plsc.VectorSubcoreMesh(core_axis_name="c", subcore_axis_name="s")

    @functools.partial(
        pl.kernel,
        out_type=jax.ShapeDtypeStruct((N_FIELDS, BATCH * EMB_DIM),
                                      jnp.float32),
        mesh=mesh,
        scratch_types=[
            pltpu.VMEM((N_FIELDS, BPW), jnp.int32),
            pltpu.VMEM((2, GLEN), jnp.int32),
            pltpu.VMEM((2, GLEN), jnp.float32),
            pltpu.SemaphoreType.DMA,
            pltpu.SemaphoreType.DMA,
        ],
        compiler_params=pltpu.CompilerParams(use_tc_tiling_on_sc=False,
                                            needs_layout_passes=False),
    )
    def k(catf_hbm, *rest):
        flats_hbm = rest[:N_FIELDS]
        out_hbm, idx_v, eidx_v, rows_v, gsem, wsem = rest[N_FIELDS:]
        wid = lax.axis_index("s") * NC + lax.axis_index("c")
        base = wid * BPW
        pltpu.sync_copy(catf_hbm.at[:, pl.ds(base, BPW)], idx_v)

        lane = jax.lax.iota(jnp.int32, 16)

        def build(f, slot):
            def group(g, _):
                v16 = idx_v[f, pl.ds(g * 16, 16)]
                for e in range(EMB_DIM):
                    plsc.store_scatter(
                        eidx_v.at[slot],
                        [lane * 16 + (g * 16 * 16 + e)],
                        v16 + (e * VOCAB),
                    )
                return 0
            lax.fori_loop(0, BPW // 16, group, 0)

        build(0, 0)
        g = pltpu.async_copy(flats_hbm[0].at[eidx_v.at[0]], rows_v.at[0],
                             gsem)
        w = None
        for f in range(N_FIELDS):
            if f + 1 < N_FIELDS:
                build(f + 1, (f + 1) % 2)
            g.wait()
            if w is not None:
                w.wait()
            w = pltpu.async_copy(
                rows_v.at[f % 2], out_hbm.at[f, pl.ds(base * EMB_DIM, GLEN)],
                wsem,
            )
            if f + 1 < N_FIELDS:
                g = pltpu.async_copy(
                    flats_hbm[f + 1].at[eidx_v.at[(f + 1) % 2]],
                    rows_v.at[(f + 1) % 2], gsem,
                )
        w.wait()

    return k(catf, *flats)


def _mlp(emb, cont, w1p, b1, w2, b2, w3, b3):
    bb = 1024
    grid = (BATCH // bb,)

    def body(emb_ref, cont_ref, w1_ref, b1_ref, w2_ref, b2_ref, w3_ref,
             b3_ref, out_ref):
        pieces = [emb_ref[f] for f in range(N_FIELDS)]
        pieces.append(cont_ref[:])
        pieces.append(jnp.zeros((bb, IN_PAD - N_FIELDS * EMB_DIM - N_CONT),
                                jnp.float32))
        x = jnp.concatenate(pieces, axis=1)
        h = jnp.dot(x, w1_ref[:], preferred_element_type=jnp.float32)
        h = jnp.maximum(h + b1_ref[:], 0.0)
        h = jnp.dot(h, w2_ref[:], preferred_element_type=jnp.float32)
        h = jnp.maximum(h + b2_ref[:], 0.0)
        out = jnp.dot(h, w3_ref[:], preferred_element_type=jnp.float32)
        out_ref[:] = out[:, 0] + b3_ref[0]

    return pl.pallas_call(
        body,
        grid=grid,
        in_specs=[
            pl.BlockSpec((N_FIELDS, bb, EMB_DIM), lambda i: (0, i, 0)),
            pl.BlockSpec((bb, N_CONT), lambda i: (i, 0)),
            pl.BlockSpec((IN_PAD, H1), lambda i: (0, 0)),
            pl.BlockSpec((H1,), lambda i: (0,)),
            pl.BlockSpec((H1, H2), lambda i: (0, 0)),
            pl.BlockSpec((H2,), lambda i: (0,)),
            pl.BlockSpec((H2, 1), lambda i: (0, 0)),
            pl.BlockSpec((1,), lambda i: (0,)),
        ],
        out_specs=pl.BlockSpec((bb,), lambda i: (i,)),
        out_shape=jax.ShapeDtypeStruct((BATCH,), jnp.float32),
        compiler_params=pltpu.CompilerParams(vmem_limit_bytes=100 * 1024 * 1024),
    )(emb, cont, w1p, b1, w2, b2, w3, b3)


def kernel(categorical, continuous, tables, W1, b1, W2, b2, W3, b3):
    flats = [t.T.reshape(-1) for t in tables]
    catf = categorical.T
    emb = _sc_gather(catf, flats).reshape(N_FIELDS, BATCH, EMB_DIM)
    w1p = jnp.zeros((IN_PAD, H1), jnp.float32).at[: W1.shape[0]].set(W1)
    return _mlp(emb, continuous, w1p, b1, W2, b2, W3, b3)


# speedup vs baseline: 4.2045x; 1.2773x over previous
import functools

import jax
import jax.numpy as jnp
from jax import lax
from jax.experimental import pallas as pl
from jax.experimental.pallas import tpu as pltpu
from jax.experimental.pallas import tpu_sc as plsc

N_FIELDS = 26
VOCAB = 100000
EMB_DIM = 16
N_CONT = 13
BATCH = 16384
H1, H2 = 64, 32

NC = 2
NS = 16
NW = NC * NS
BPW = BATCH // NW

IN_PAD = 432
GLEN = BPW * EMB_DIM


def _sc_gather(catf, flats):
    nf = len(flats)
    mesh = plsc.VectorSubcoreMesh(core_axis_name="c", subcore_axis_name="s")

    @functools.partial(
        pl.kernel,
        out_type=jax.ShapeDtypeStruct((nf, BATCH * EMB_DIM),
                                      jnp.float32),
        mesh=mesh,
        scratch_types=[
            pltpu.VMEM((nf, BPW), jnp.int32),
            pltpu.VMEM((2, GLEN), jnp.int32),
            pltpu.VMEM((2, GLEN), jnp.float32),
            pltpu.SemaphoreType.DMA,
            pltpu.SemaphoreType.DMA,
        ],
        compiler_params=pltpu.CompilerParams(use_tc_tiling_on_sc=False,
                                            needs_layout_passes=False),
    )
    def k(catf_hbm, *rest):
        flats_hbm = rest[:nf]
        out_hbm, idx_v, eidx_v, rows_v, gsem, wsem = rest[nf:]
        wid = lax.axis_index("s") * NC + lax.axis_index("c")
        base = wid * BPW
        pltpu.sync_copy(catf_hbm.at[:, pl.ds(base, BPW)], idx_v)

        lane = jax.lax.iota(jnp.int32, 16)

        def build(f, slot):
            def group(g, _):
                v16 = idx_v[f, pl.ds(g * 16, 16)]
                for e in range(EMB_DIM):
                    plsc.store_scatter(
                        eidx_v.at[slot],
                        [lane * 16 + (g * 16 * 16 + e)],
                        v16 + (e * VOCAB),
                    )
                return 0
            lax.fori_loop(0, BPW // 16, group, 0)

        build(0, 0)
        g = pltpu.async_copy(flats_hbm[0].at[eidx_v.at[0]], rows_v.at[0],
                             gsem)
        w = None
        for f in range(nf):
            if f + 1 < nf:
                build(f + 1, (f + 1) % 2)
            g.wait()
            if w is not None:
                w.wait()
            w = pltpu.async_copy(
                rows_v.at[f % 2], out_hbm.at[f, pl.ds(base * EMB_DIM, GLEN)],
                wsem,
            )
            if f + 1 < nf:
                g = pltpu.async_copy(
                    flats_hbm[f + 1].at[eidx_v.at[(f + 1) % 2]],
                    rows_v.at[(f + 1) % 2], gsem,
                )
        w.wait()

    return k(catf, *flats)


def _mlp(embs, cont, w1p, b1, w2, b2, w3, b3):
    bb = 1024
    grid = (BATCH // bb,)
    nfs = [e.shape[0] for e in embs]

    def body(*refs):
        emb_refs = refs[: len(nfs)]
        (cont_ref, w1_ref, b1_ref, w2_ref, b2_ref, w3_ref, b3_ref,
         out_ref) = refs[len(nfs):]
        pieces = []
        for r, n in zip(emb_refs, nfs):
            pieces.extend(r[f] for f in range(n))
        pieces.append(cont_ref[:])
        pieces.append(jnp.zeros((bb, IN_PAD - N_FIELDS * EMB_DIM - N_CONT),
                                jnp.float32))
        x = jnp.concatenate(pieces, axis=1)
        h = jnp.dot(x, w1_ref[:], preferred_element_type=jnp.float32)
        h = jnp.maximum(h + b1_ref[:], 0.0)
        h = jnp.dot(h, w2_ref[:], preferred_element_type=jnp.float32)
        h = jnp.maximum(h + b2_ref[:], 0.0)
        out = jnp.dot(h, w3_ref[:], preferred_element_type=jnp.float32)
        out_ref[:] = out[:, 0] + b3_ref[0]

    return pl.pallas_call(
        body,
        grid=grid,
        in_specs=[
            pl.BlockSpec((n, bb, EMB_DIM), lambda i: (0, i, 0)) for n in nfs
        ] + [
            pl.BlockSpec((bb, N_CONT), lambda i: (i, 0)),
            pl.BlockSpec((IN_PAD, H1), lambda i: (0, 0)),
            pl.BlockSpec((H1,), lambda i: (0,)),
            pl.BlockSpec((H1, H2), lambda i: (0, 0)),
            pl.BlockSpec((H2,), lambda i: (0,)),
            pl.BlockSpec((H2, 1), lambda i: (0, 0)),
            pl.BlockSpec((1,), lambda i: (0,)),
        ],
        out_specs=pl.BlockSpec((bb,), lambda i: (i,)),
        out_shape=jax.ShapeDtypeStruct((BATCH,), jnp.float32),
        compiler_params=pltpu.CompilerParams(vmem_limit_bytes=100 * 1024 * 1024),
    )(*embs, cont, w1p, b1, w2, b2, w3, b3)


def kernel(categorical, continuous, tables, W1, b1, W2, b2, W3, b3):
    flats = [t.T.reshape(-1) for t in tables]
    catf = categorical.T
    splits = (7, 14, 20, 26)
    lo = 0
    parts = []
    for hi in splits:
        nf = hi - lo
        parts.append(
            _sc_gather(catf[lo:hi], flats[lo:hi]).reshape(nf, BATCH, EMB_DIM)
        )
        lo = hi
    w1p = jnp.zeros((IN_PAD, H1), jnp.float32).at[: W1.shape[0]].set(W1)
    return _mlp(parts, continuous, w1p, b1, W2, b2, W3, b3)


# speedup vs baseline: 5.1739x; 1.2306x over previous
import functools

import jax
import jax.numpy as jnp
from jax import lax
from jax.experimental import pallas as pl
from jax.experimental.pallas import tpu as pltpu
from jax.experimental.pallas import tpu_sc as plsc

N_FIELDS = 26
VOCAB = 100000
EMB_DIM = 16
N_CONT = 13
BATCH = 16384
H1, H2 = 64, 32

NC = 2
NS = 16
NW = NC * NS
BPW = BATCH // NW

IN_PAD = 432
GLEN = BPW * EMB_DIM


def _sc_gather(catf, flats):
    nf = len(flats)
    mesh = plsc.VectorSubcoreMesh(core_axis_name="c", subcore_axis_name="s")

    @functools.partial(
        pl.kernel,
        out_type=jax.ShapeDtypeStruct((nf, BATCH * EMB_DIM),
                                      jnp.float32),
        mesh=mesh,
        scratch_types=[
            pltpu.VMEM((nf, BPW), jnp.int32),
            pltpu.VMEM((2, GLEN), jnp.int32),
            pltpu.VMEM((2, GLEN), jnp.float32),
            pltpu.SemaphoreType.DMA,
            pltpu.SemaphoreType.DMA,
        ],
        compiler_params=pltpu.CompilerParams(use_tc_tiling_on_sc=False,
                                            needs_layout_passes=False),
    )
    def k(catf_hbm, *rest):
        flats_hbm = rest[:nf]
        out_hbm, idx_v, eidx_v, rows_v, gsem, wsem = rest[nf:]
        wid = lax.axis_index("s") * NC + lax.axis_index("c")
        base = wid * BPW
        pltpu.sync_copy(catf_hbm.at[:, pl.ds(base, BPW)], idx_v)

        lane = jax.lax.iota(jnp.int32, 16)

        def build(f, slot):
            def group(g, _):
                v16 = idx_v[f, pl.ds(g * 16, 16)]
                for e in range(EMB_DIM):
                    plsc.store_scatter(
                        eidx_v.at[slot],
                        [lane * 16 + (g * 16 * 16 + e)],
                        v16 + (e * VOCAB),
                    )
                return 0
            lax.fori_loop(0, BPW // 16, group, 0)

        build(0, 0)
        g = pltpu.async_copy(flats_hbm[0].at[eidx_v.at[0]], rows_v.at[0],
                             gsem)
        w = None
        for f in range(nf):
            if f + 1 < nf:
                build(f + 1, (f + 1) % 2)
            g.wait()
            if w is not None:
                w.wait()
            w = pltpu.async_copy(
                rows_v.at[f % 2], out_hbm.at[f, pl.ds(base * EMB_DIM, GLEN)],
                wsem,
            )
            if f + 1 < nf:
                g = pltpu.async_copy(
                    flats_hbm[f + 1].at[eidx_v.at[(f + 1) % 2]],
                    rows_v.at[(f + 1) % 2], gsem,
                )
        w.wait()

    return k(catf, *flats)


def _mlp_packed(embs, cont_p, w1x, b1x, w2x, b2x, w3x, b3):
    bbp = 256
    grid = (BATCH // 8 // bbp,)
    nfs = [e.shape[0] for e in embs]

    def body(*refs):
        emb_refs = refs[: len(nfs)]
        (cont_ref, w1_ref, b1_ref, w2_ref, b2_ref, w3_ref, b3_ref,
         out_ref) = refs[len(nfs):]
        pieces = []
        for r, n in zip(emb_refs, nfs):
            pieces.extend(r[f] for f in range(n))
        pieces.append(cont_ref[:])
        x = jnp.concatenate(pieces, axis=1)
        h = jnp.dot(x, w1_ref[:], preferred_element_type=jnp.float32)
        h = jnp.maximum(h + b1_ref[:], 0.0)
        h = jnp.dot(h, w2_ref[:], preferred_element_type=jnp.float32)
        h = jnp.maximum(h + b2_ref[:], 0.0)
        out = jnp.dot(h, w3_ref[:], preferred_element_type=jnp.float32)
        out_ref[:] = out + b3_ref[0]

    kin = N_FIELDS * 128 + 104

    return pl.pallas_call(
        body,
        grid=grid,
        in_specs=[
            pl.BlockSpec((n, bbp, 128), lambda i: (0, i, 0)) for n in nfs
        ] + [
            pl.BlockSpec((bbp, 104), lambda i: (i, 0)),
            pl.BlockSpec((kin, 8 * H1), lambda i: (0, 0)),
            pl.BlockSpec((8 * H1,), lambda i: (0,)),
            pl.BlockSpec((8 * H1, 8 * H2), lambda i: (0, 0)),
            pl.BlockSpec((8 * H2,), lambda i: (0,)),
            pl.BlockSpec((8 * H2, 8), lambda i: (0, 0)),
            pl.BlockSpec((1,), lambda i: (0,)),
        ],
        out_specs=pl.BlockSpec((bbp, 8), lambda i: (i, 0)),
        out_shape=jax.ShapeDtypeStruct((BATCH // 8, 8), jnp.float32),
        compiler_params=pltpu.CompilerParams(vmem_limit_bytes=100 * 1024 * 1024),
    )(*embs, cont_p, w1x, b1x, w2x, b2x, w3x, b3)


def kernel(categorical, continuous, tables, W1, b1, W2, b2, W3, b3):
    flats = [t.T.reshape(-1) for t in tables]
    catf = categorical.T
    splits = (7, 14, 20, 26)
    lo = 0
    parts = []
    for hi in splits:
        nf = hi - lo
        parts.append(
            _sc_gather(catf[lo:hi], flats[lo:hi]).reshape(
                nf, BATCH // 8, 128
            )
        )
        lo = hi
    cont_p = continuous.reshape(BATCH // 8, 8 * N_CONT)
    i8 = jnp.eye(8, dtype=jnp.float32)
    w1r = W1[: N_FIELDS * EMB_DIM].reshape(N_FIELDS, EMB_DIM, H1)
    w1x = jnp.concatenate(
        [
            jnp.einsum("feh,jk->fjekh", w1r, i8).reshape(
                N_FIELDS * 128, 8 * H1
            ),
            jnp.einsum("ch,jk->jckh", W1[N_FIELDS * EMB_DIM:], i8).reshape(
                8 * N_CONT, 8 * H1
            ),
        ],
        axis=0,
    )
    b1x = jnp.tile(b1, 8)
    w2x = jnp.einsum("km,jl->jklm", W2, i8).reshape(8 * H1, 8 * H2)
    b2x = jnp.tile(b2, 8)
    w3x = jnp.einsum("mo,jl->jmlo", W3, i8).reshape(8 * H2, 8)
    out = _mlp_packed(parts, cont_p, w1x, b1x, w2x, b2x, w3x, b3)
    return out.reshape(BATCH)


# speedup vs baseline: 5.2565x; 1.0160x over previous
import functools

import jax
import jax.numpy as jnp
from jax import lax
from jax.experimental import pallas as pl
from jax.experimental.pallas import tpu as pltpu
from jax.experimental.pallas import tpu_sc as plsc

N_FIELDS = 26
VOCAB = 100000
EMB_DIM = 16
N_CONT = 13
BATCH = 16384
H1, H2 = 64, 32

NC = 2
NS = 16
NW = NC * NS
BPW = BATCH // NW

IN_PAD = 432
GLEN = BPW * EMB_DIM


def _sc_gather(catf, flats):
    nf = len(flats)
    mesh = plsc.VectorSubcoreMesh(core_axis_name="c", subcore_axis_name="s")

    @functools.partial(
        pl.kernel,
        out_type=jax.ShapeDtypeStruct((nf, BATCH * EMB_DIM),
                                      jnp.float32),
        mesh=mesh,
        scratch_types=[
            pltpu.VMEM((nf, BPW), jnp.int32),
            pltpu.VMEM((4, GLEN), jnp.int32),
            pltpu.VMEM((4, GLEN), jnp.float32),
            pltpu.SemaphoreType.DMA,
            pltpu.SemaphoreType.DMA,
            pltpu.SemaphoreType.DMA,
            pltpu.SemaphoreType.DMA,
            pltpu.SemaphoreType.DMA,
        ],
        compiler_params=pltpu.CompilerParams(use_tc_tiling_on_sc=False,
                                            needs_layout_passes=False),
    )
    def k(catf_hbm, *rest):
        flats_hbm = rest[:nf]
        (out_hbm, idx_v, eidx_v, rows_v,
         gsem0, gsem1, gsem2, gsem3, wsem) = rest[nf:]
        gsems = (gsem0, gsem1, gsem2, gsem3)
        wid = lax.axis_index("s") * NC + lax.axis_index("c")
        base = wid * BPW
        pltpu.sync_copy(catf_hbm.at[:, pl.ds(base, BPW)], idx_v)

        lane = jax.lax.iota(jnp.int32, 16)

        def build(f, slot):
            def group(g, _):
                v16 = idx_v[f, pl.ds(g * 16, 16)]
                for e in range(EMB_DIM):
                    plsc.store_scatter(
                        eidx_v.at[slot],
                        [lane * 16 + (g * 16 * 16 + e)],
                        v16 + (e * VOCAB),
                    )
                return 0
            lax.fori_loop(0, BPW // 16, group, 0)

        depth = 3
        gs = [None] * nf
        for j in range(min(depth, nf)):
            build(j, j % 4)
            gs[j] = pltpu.async_copy(
                flats_hbm[j].at[eidx_v.at[j % 4]], rows_v.at[j % 4],
                gsems[j % 4],
            )
        w = None
        for f in range(nf):
            gs[f].wait()
            if w is not None:
                w.wait()
            w = pltpu.async_copy(
                rows_v.at[f % 4], out_hbm.at[f, pl.ds(base * EMB_DIM, GLEN)],
                wsem,
            )
            nxt = f + depth
            if nxt < nf:
                build(nxt, nxt % 4)
                gs[nxt] = pltpu.async_copy(
                    flats_hbm[nxt].at[eidx_v.at[nxt % 4]], rows_v.at[nxt % 4],
                    gsems[nxt % 4],
                )
        w.wait()

    return k(catf, *flats)


def _mlp_packed(embs, cont_p, w1x, b1x, w2x, b2x, w3x, b3):
    bbp = 256
    grid = (BATCH // 8 // bbp,)
    nfs = [e.shape[0] for e in embs]

    def body(*refs):
        emb_refs = refs[: len(nfs)]
        (cont_ref, w1_ref, b1_ref, w2_ref, b2_ref, w3_ref, b3_ref,
         out_ref) = refs[len(nfs):]
        pieces = []
        for r, n in zip(emb_refs, nfs):
            pieces.extend(r[f] for f in range(n))
        pieces.append(cont_ref[:])
        x = jnp.concatenate(pieces, axis=1)
        h = jnp.dot(x, w1_ref[:], preferred_element_type=jnp.float32)
        h = jnp.maximum(h + b1_ref[:], 0.0)
        h = jnp.dot(h, w2_ref[:], preferred_element_type=jnp.float32)
        h = jnp.maximum(h + b2_ref[:], 0.0)
        out = jnp.dot(h, w3_ref[:], preferred_element_type=jnp.float32)
        out_ref[:] = out + b3_ref[0]

    kin = N_FIELDS * 128 + 104

    return pl.pallas_call(
        body,
        grid=grid,
        in_specs=[
            pl.BlockSpec((n, bbp, 128), lambda i: (0, i, 0)) for n in nfs
        ] + [
            pl.BlockSpec((bbp, 104), lambda i: (i, 0)),
            pl.BlockSpec((kin, 8 * H1), lambda i: (0, 0)),
            pl.BlockSpec((8 * H1,), lambda i: (0,)),
            pl.BlockSpec((8 * H1, 8 * H2), lambda i: (0, 0)),
            pl.BlockSpec((8 * H2,), lambda i: (0,)),
            pl.BlockSpec((8 * H2, 8), lambda i: (0, 0)),
            pl.BlockSpec((1,), lambda i: (0,)),
        ],
        out_specs=pl.BlockSpec((bbp, 8), lambda i: (i, 0)),
        out_shape=jax.ShapeDtypeStruct((BATCH // 8, 8), jnp.float32),
        compiler_params=pltpu.CompilerParams(vmem_limit_bytes=100 * 1024 * 1024),
    )(*embs, cont_p, w1x, b1x, w2x, b2x, w3x, b3)


def kernel(categorical, continuous, tables, W1, b1, W2, b2, W3, b3):
    flats = [t.T.reshape(-1) for t in tables]
    catf = categorical.T
    splits = (1, 6, 13, 19, 26)
    lo = 0
    parts = []
    for hi in splits:
        nf = hi - lo
        parts.append(
            _sc_gather(catf[lo:hi], flats[lo:hi]).reshape(
                nf, BATCH // 8, 128
            )
        )
        lo = hi
    cont_p = continuous.reshape(BATCH // 8, 8 * N_CONT)
    i8 = jnp.eye(8, dtype=jnp.float32)
    w1r = W1[: N_FIELDS * EMB_DIM].reshape(N_FIELDS, EMB_DIM, H1)
    w1x = jnp.concatenate(
        [
            jnp.einsum("feh,jk->fjekh", w1r, i8).reshape(
                N_FIELDS * 128, 8 * H1
            ),
            jnp.einsum("ch,jk->jckh", W1[N_FIELDS * EMB_DIM:], i8).reshape(
                8 * N_CONT, 8 * H1
            ),
        ],
        axis=0,
    )
    b1x = jnp.tile(b1, 8)
    w2x = jnp.einsum("km,jl->jklm", W2, i8).reshape(8 * H1, 8 * H2)
    b2x = jnp.tile(b2, 8)
    w3x = jnp.einsum("mo,jl->jmlo", W3, i8).reshape(8 * H2, 8)
    out = _mlp_packed(parts, cont_p, w1x, b1x, w2x, b2x, w3x, b3)
    return out.reshape(BATCH)


# speedup vs baseline: 5.2659x; 1.0018x over previous
import functools

import jax
import jax.numpy as jnp
from jax import lax
from jax.experimental import pallas as pl
from jax.experimental.pallas import tpu as pltpu
from jax.experimental.pallas import tpu_sc as plsc

N_FIELDS = 26
VOCAB = 100000
EMB_DIM = 16
N_CONT = 13
BATCH = 16384
H1, H2 = 64, 32

NC = 2
NS = 16
NW = NC * NS
BPW = BATCH // NW

GLEN = BPW * EMB_DIM


def _sc_gather(catf, flats):
    nf = len(flats)
    mesh = plsc.VectorSubcoreMesh(core_axis_name="c", subcore_axis_name="s")

    @functools.partial(
        pl.kernel,
        out_type=jax.ShapeDtypeStruct((nf, BATCH * EMB_DIM),
                                      jnp.float32),
        mesh=mesh,
        scratch_types=[
            pltpu.VMEM((nf, BPW), jnp.int32),
            pltpu.VMEM((4, GLEN), jnp.int32),
            pltpu.VMEM((4, GLEN), jnp.float32),
            pltpu.SemaphoreType.DMA,
            pltpu.SemaphoreType.DMA,
            pltpu.SemaphoreType.DMA,
            pltpu.SemaphoreType.DMA,
            pltpu.SemaphoreType.DMA,
        ],
        compiler_params=pltpu.CompilerParams(use_tc_tiling_on_sc=False,
                                            needs_layout_passes=False),
    )
    def k(catf_hbm, *rest):
        flats_hbm = rest[:nf]
        (out_hbm, idx_v, eidx_v, rows_v,
         gsem0, gsem1, gsem2, gsem3, wsem) = rest[nf:]
        gsems = (gsem0, gsem1, gsem2, gsem3)
        wid = lax.axis_index("s") * NC + lax.axis_index("c")
        base = wid * BPW
        pltpu.sync_copy(catf_hbm.at[:, pl.ds(base, BPW)], idx_v)

        lane = jax.lax.iota(jnp.int32, 16)

        def build(f, slot):
            def group(g, _):
                v16 = idx_v[f, pl.ds(g * 16, 16)]
                for e in range(EMB_DIM):
                    plsc.store_scatter(
                        eidx_v.at[slot],
                        [lane * 16 + (g * 16 * 16 + e)],
                        v16 + (e * VOCAB),
                    )
                return 0
            lax.fori_loop(0, BPW // 16, group, 0)

        depth = 3
        gs = [None] * nf
        for j in range(min(depth, nf)):
            build(j, j % 4)
            gs[j] = pltpu.async_copy(
                flats_hbm[j].at[eidx_v.at[j % 4]], rows_v.at[j % 4],
                gsems[j % 4],
            )
        w = None
        for f in range(nf):
            gs[f].wait()
            if w is not None:
                w.wait()
            w = pltpu.async_copy(
                rows_v.at[f % 4], out_hbm.at[f, pl.ds(base * EMB_DIM, GLEN)],
                wsem,
            )
            nxt = f + depth
            if nxt < nf:
                build(nxt, nxt % 4)
                gs[nxt] = pltpu.async_copy(
                    flats_hbm[nxt].at[eidx_v.at[nxt % 4]], rows_v.at[nxt % 4],
                    gsems[nxt % 4],
                )
        w.wait()

    return k(catf, *flats)


def _mlp_packed(embs, cont_p, w1x, b1x, w2x, b2x, w3x, b3):
    bbp = 256
    grid = (BATCH // 8 // bbp,)
    nfs = [e.shape[0] for e in embs]

    def body(*refs):
        emb_refs = refs[: len(nfs)]
        (cont_ref, w1_ref, b1_ref, w2_ref, b2_ref, w3_ref, b3_ref,
         out_ref) = refs[len(nfs):]
        pieces = []
        for r, n in zip(emb_refs, nfs):
            pieces.extend(r[f] for f in range(n))
        pieces.append(cont_ref[:])
        x = jnp.concatenate(pieces, axis=1)
        h = jnp.dot(x, w1_ref[:], preferred_element_type=jnp.float32)
        h = jnp.maximum(h + b1_ref[:], 0.0)
        h = jnp.dot(h, w2_ref[:], preferred_element_type=jnp.float32)
        h = jnp.maximum(h + b2_ref[:], 0.0)
        out = jnp.dot(h, w3_ref[:], preferred_element_type=jnp.float32)
        out_ref[:] = out + b3_ref[0]

    kin = N_FIELDS * 128 + 104

    return pl.pallas_call(
        body,
        grid=grid,
        in_specs=[
            pl.BlockSpec((n, bbp, 128), lambda i: (0, i, 0)) for n in nfs
        ] + [
            pl.BlockSpec((bbp, 104), lambda i: (i, 0)),
            pl.BlockSpec((kin, 8 * H1), lambda i: (0, 0)),
            pl.BlockSpec((8 * H1,), lambda i: (0,)),
            pl.BlockSpec((8 * H1, 8 * H2), lambda i: (0, 0)),
            pl.BlockSpec((8 * H2,), lambda i: (0,)),
            pl.BlockSpec((8 * H2, 8), lambda i: (0, 0)),
            pl.BlockSpec((1,), lambda i: (0,)),
        ],
        out_specs=pl.BlockSpec((bbp, 8), lambda i: (i, 0)),
        out_shape=jax.ShapeDtypeStruct((BATCH // 8, 8), jnp.float32),
        compiler_params=pltpu.CompilerParams(vmem_limit_bytes=100 * 1024 * 1024),
    )(*embs, cont_p, w1x, b1x, w2x, b2x, w3x, b3)


def kernel(categorical, continuous, tables, W1, b1, W2, b2, W3, b3):
    flats = [t.T.reshape(-1) for t in tables]
    catf = categorical.T
    splits = (1, 6, 13, 19, 26)
    lo = 0
    parts = []
    for hi in splits:
        nf = hi - lo
        parts.append(
            _sc_gather(catf[lo:hi], flats[lo:hi]).reshape(
                nf, BATCH // 8, 128
            )
        )
        lo = hi
    cont_p = continuous.reshape(BATCH // 8, 8 * N_CONT)
    i8 = jnp.eye(8, dtype=jnp.float32)
    w1r = W1[: N_FIELDS * EMB_DIM].reshape(N_FIELDS, EMB_DIM, H1)
    w1x = jnp.concatenate(
        [
            jnp.einsum("feh,jk->fjekh", w1r, i8).reshape(
                N_FIELDS * 128, 8 * H1
            ),
            jnp.einsum("ch,jk->jckh", W1[N_FIELDS * EMB_DIM:], i8).reshape(
                8 * N_CONT, 8 * H1
            ),
        ],
        axis=0,
    )
    b1x = jnp.tile(b1, 8)
    w2x = jnp.einsum("km,jl->jklm", W2, i8).reshape(8 * H1, 8 * H2)
    b2x = jnp.tile(b2, 8)
    w3x = jnp.einsum("mo,jl->jmlo", W3, i8).reshape(8 * H2, 8)
    out = _mlp_packed(parts, cont_p, w1x, b1x, w2x, b2x, w3x, b3)
    return out.reshape(BATCH)
